# Initial kernel scaffold; baseline (speedup 1.0000x reference)
import jax
import jax.numpy as jnp
from jax.experimental import pallas as pl


def kernel(feat_drug, feat_disease, edge_index_drug_drug, edge_index_drug_disease, edge_index_disease_drug, edge_index_disease_disease, mp_ins, W_lin_drug, b_lin_drug, W_lin_dis, b_lin_dis, Wg, bg, a_prelu, W_res_drug, b_res_drug, W_res_dis, b_res_dis, W_dd, W_dr, W_half_drug, W_half_dis, W_mil1, b_mil1, W_mil2, W_ins, W_mlp):
    raise NotImplementedError("write your pallas kernel here")


# speedup vs baseline: 4.0699x; 4.0699x over previous
import functools

import jax
import jax.numpy as jnp
from jax import lax
from jax.experimental import pallas as pl
from jax.experimental.pallas import tpu as pltpu
from jax.experimental.pallas import tpu_sc as plsc

N = 10000
D = 256
HALF = 128
E = 320000
CH = 128
NCH = 2560
EPAD = NCH * CH
APAD = 10240
STRIP = APAD // 16
B = 1024
BAG = 64
KTOP = 5
NS = 16
NC = 2
TPC = NCH // NS
MCH = (B * BAG) // CH // (NC * NS)

F32 = jnp.float32
_SC_CACHE = {}


def _sc_mesh():
    return plsc.VectorSubcoreMesh(core_axis_name="c", subcore_axis_name="s")


def _sc_degrees(*args):
    if "deg" not in _SC_CACHE:
        _SC_CACHE["deg"] = functools.partial(
            pl.kernel,
            mesh=_sc_mesh(),
            out_type=[jax.ShapeDtypeStruct((APAD,), F32) for _ in range(8)],
            scratch_types=[
                pltpu.VMEM((TPC, CH), jnp.int32),
                pltpu.VMEM((CH,), F32),
                pltpu.VMEM((STRIP,), F32),
                pltpu.VMEM_SHARED((APAD,), F32),
                pltpu.VMEM_SHARED((APAD,), F32),
                pltpu.VMEM_SHARED((APAD,), F32),
                pltpu.VMEM_SHARED((APAD,), F32),
            ],
        )(_sc_degrees_body)
    return _SC_CACHE["deg"](*args)


def _sc_degrees_body(ones_hbm, zeros_hbm,
                dd_s, dd_d, ds_s, ds_d, sd_s, sd_d, ss_s, ss_d,
                o0, o1, o2, o3, o4, o5, o6, o7,
                ibuf, ones_v, zer_v, h0, h1, h2, h3):
    c = lax.axis_index("c")
    s = lax.axis_index("s")
    pltpu.sync_copy(ones_hbm, ones_v)
    pltpu.sync_copy(zeros_hbm, zer_v)
    hists = [h0, h1, h2, h3]
    for h in hists:
        pltpu.sync_copy(zer_v, h.at[pl.ds(s * STRIP, STRIP)])
    plsc.subcore_barrier()

    def hist_pass(idx_hbm, hist):
        pltpu.sync_copy(idx_hbm.at[pl.ds(s * TPC, TPC)], ibuf)

        def body(j, carry):
            pltpu.sync_copy(ones_v, hist.at[ibuf.at[j]], add=True)
            return carry

        lax.fori_loop(0, TPC, body, 0)

    @pl.when(c == 0)
    def _():
        for idx, h in zip([dd_s, dd_d, ds_s, ds_d], hists):
            hist_pass(idx, h)

    @pl.when(c == 1)
    def _():
        for idx, h in zip([sd_s, sd_d, ss_s, ss_d], hists):
            hist_pass(idx, h)

    plsc.subcore_barrier()
    outs01 = [o0, o1, o2, o3]
    outs23 = [o4, o5, o6, o7]

    @pl.when(c == 0)
    def _():
        for h, o in zip(hists, outs01):
            pltpu.sync_copy(h.at[pl.ds(s * STRIP, STRIP)],
                            o.at[pl.ds(s * STRIP, STRIP)])

    @pl.when(c == 1)
    def _():
        for h, o in zip(hists, outs23):
            pltpu.sync_copy(h.at[pl.ds(s * STRIP, STRIP)],
                            o.at[pl.ds(s * STRIP, STRIP)])


def _sc_agg(*args):
    if "agg" not in _SC_CACHE:
        _SC_CACHE["agg"] = functools.partial(
            pl.kernel,
            mesh=_sc_mesh(),
            out_type=[jax.ShapeDtypeStruct((APAD, HALF), F32) for _ in range(8)],
            scratch_types=[
                pltpu.VMEM((TPC // 2, CH), jnp.int32),
                pltpu.VMEM((TPC // 2, CH), jnp.int32),
                pltpu.VMEM((CH, HALF), F32),
                pltpu.SemaphoreType.DMA,
                pltpu.VMEM_SHARED((APAD, HALF), F32),
            ],
        )(_sc_agg_body)
    return _SC_CACHE["agg"](*args)


def _sc_agg_body(zeros_hbm,
            src0, dst0, src1, dst1, src2, dst2, src3, dst3,
            tA0, tB0, tA1, tB1, tA2, tB2, tA3, tB3,
            oA0, oB0, oA1, oB1, oA2, oB2, oA3, oB3,
            sbuf, dbuf, u0, sem0, acc):
    c = lax.axis_index("c")
    s = lax.axis_index("s")
    edges = [(src0, dst0), (src1, dst1), (src2, dst2), (src3, dst3)]
    tabs = [(tA0, tB0), (tA1, tB1), (tA2, tB2), (tA3, tB3)]
    outs = [(oA0, oB0), (oA1, oB1), (oA2, oB2), (oA3, oB3)]
    hc = TPC // 2
    for r in range(4):
        pltpu.sync_copy(zeros_hbm, u0)
        for k2 in range(STRIP // CH):
            pltpu.sync_copy(u0, acc.at[pl.ds(s * STRIP + k2 * CH, CH)])
        plsc.subcore_barrier()
        srcp, dstp = edges[r]

        def process(tbl, srcp=srcp, dstp=dstp):
            for hv in range(2):
                pltpu.sync_copy(srcp.at[pl.ds(s * TPC + hv * hc, hc)], sbuf)
                pltpu.sync_copy(dstp.at[pl.ds(s * TPC + hv * hc, hc)], dbuf)

                def body(j, carry):
                    pltpu.async_copy(tbl.at[sbuf.at[j]], u0, sem0).wait()
                    pltpu.sync_copy(u0, acc.at[dbuf.at[j]], add=True)
                    return carry

                lax.fori_loop(0, hc, body, 0)

        tA, tB = tabs[r]

        @pl.when(c == 0)
        def _():
            process(tA)

        @pl.when(c == 1)
        def _():
            process(tB)

        plsc.subcore_barrier()
        oA, oB = outs[r]

        @pl.when(c == 0)
        def _():
            pltpu.sync_copy(acc.at[pl.ds(s * STRIP, STRIP)],
                            oA.at[pl.ds(s * STRIP, STRIP)])

        @pl.when(c == 1)
        def _():
            pltpu.sync_copy(acc.at[pl.ds(s * STRIP, STRIP)],
                            oB.at[pl.ds(s * STRIP, STRIP)])


def _sc_meta(*args):
    if "meta" not in _SC_CACHE:
        _SC_CACHE["meta"] = functools.partial(
            pl.kernel,
            mesh=_sc_mesh(),
            out_type=[jax.ShapeDtypeStruct((B * BAG, HALF), F32) for _ in range(8)],
            scratch_types=[
                pltpu.VMEM((MCH, CH), jnp.int32),
                pltpu.VMEM((CH, HALF), F32),
                pltpu.SemaphoreType.DMA,
            ],
        )(_sc_meta_body)
    return _SC_CACHE["meta"](*args)


def _sc_meta_body(idx0, idx1, idx2, idx3, fdA, fdB, fsA, fsB,
             g0A, g0B, g1A, g1B, g2A, g2B, g3A, g3B,
             ibuf, u0, sem0):
    c = lax.axis_index("c")
    s = lax.axis_index("s")
    wid = s * NC + c
    jobs = [(idx0, fdA, g0A), (idx0, fdB, g0B),
            (idx1, fdA, g1A), (idx1, fdB, g1B),
            (idx2, fsA, g2A), (idx2, fsB, g2B),
            (idx3, fsA, g3A), (idx3, fsB, g3B)]
    for idx, tbl, out in jobs:
        pltpu.sync_copy(idx.at[pl.ds(wid * MCH, MCH)], ibuf)

        def body(j, carry, tbl=tbl, out=out):
            pltpu.async_copy(tbl.at[ibuf.at[j]], u0, sem0).wait()
            pltpu.sync_copy(u0, out.at[pl.ds((wid * MCH + j) * CH, CH)])
            return carry

        lax.fori_loop(0, MCH, body, 0)


_RB = 1000


def _dot(a, b):
    return jnp.dot(a, b, preferred_element_type=F32)


def _full(shape):
    return pl.BlockSpec(shape, lambda i: (0,) * len(shape))


def _rows(shape):
    return pl.BlockSpec(shape, lambda i: (i,) + (0,) * (len(shape) - 1))


def _nrm(deg):
    return lax.rsqrt(jnp.maximum(deg, 1.0))


def _pre_body(fd, fs, wd, ws, bd, bs, g_dd, g_ds, g_sd, g_ss,
              hd_o, hs_o, xdda, xddb, xdsa, xdsb, xsda, xsdb, xssa, xssb):
    hd = _dot(fd[...], wd[...]) + bd[...]
    hs = _dot(fs[...], ws[...]) + bs[...]
    hd_o[...] = hd
    hs_o[...] = hs
    xdd = hd * _nrm(g_dd[...])
    xds = hd * _nrm(g_ds[...])
    xsd = hs * _nrm(g_sd[...])
    xss = hs * _nrm(g_ss[...])
    xdda[...] = xdd[:, :HALF]
    xddb[...] = xdd[:, HALF:]
    xdsa[...] = xds[:, :HALF]
    xdsb[...] = xds[:, HALF:]
    xsda[...] = xsd[:, :HALF]
    xsdb[...] = xsd[:, HALF:]
    xssa[...] = xss[:, :HALF]
    xssb[...] = xss[:, HALF:]


def _tc_pre(fd, fs, wd, ws, bd, bs, g_dd, g_ds, g_sd, g_ss):
    nblk = N // _RB
    outs = ([jax.ShapeDtypeStruct((N, D), F32)] * 2
            + [jax.ShapeDtypeStruct((N, HALF), F32)] * 8)
    return pl.pallas_call(
        _pre_body,
        grid=(nblk,),
        in_specs=[_rows((_RB, D)), _rows((_RB, D)),
                  _full((D, D)), _full((D, D)),
                  _full((1, D)), _full((1, D)),
                  _rows((_RB, 1)), _rows((_RB, 1)),
                  _rows((_RB, 1)), _rows((_RB, 1))],
        out_specs=[_rows((_RB, D))] * 2 + [_rows((_RB, HALF))] * 8,
        out_shape=outs,
    )(fd, fs, wd, ws, bd, bs, g_dd, g_ds, g_sd, g_ss)


def _post_body_next(adda, addb, asda, asdb, adsa, adsb, assa, assb,
                    w0a, w0b, w1a, w1b, w2a, w2b, w3a, w3b,
                    b01, b23, gi_dd, gi_sd, gi_ds, gi_ss, alpha,
                    g_dd, g_ds, g_sd, g_ss,
                    hd_o, hs_o, xdda, xddb, xdsa, xdsb, xsda, xsdb, xssa, xssb):
    a = alpha[0, 0]
    dn = (_dot(adda[...], w0a[...]) + _dot(addb[...], w0b[...])) * _nrm(gi_dd[...])
    dn = dn + (_dot(asda[...], w1a[...]) + _dot(asdb[...], w1b[...])) * _nrm(gi_sd[...])
    dn = dn + b01[...]
    hd = jnp.where(dn > 0, dn, a * dn)
    sn = (_dot(adsa[...], w2a[...]) + _dot(adsb[...], w2b[...])) * _nrm(gi_ds[...])
    sn = sn + (_dot(assa[...], w3a[...]) + _dot(assb[...], w3b[...])) * _nrm(gi_ss[...])
    sn = sn + b23[...]
    hs = jnp.where(sn > 0, sn, a * sn)
    hd_o[...] = hd
    hs_o[...] = hs
    xdd = hd * _nrm(g_dd[...])
    xds = hd * _nrm(g_ds[...])
    xsd = hs * _nrm(g_sd[...])
    xss = hs * _nrm(g_ss[...])
    xdda[...] = xdd[:, :HALF]
    xddb[...] = xdd[:, HALF:]
    xdsa[...] = xds[:, :HALF]
    xdsb[...] = xds[:, HALF:]
    xsda[...] = xsd[:, :HALF]
    xsdb[...] = xsd[:, HALF:]
    xssa[...] = xss[:, :HALF]
    xssb[...] = xss[:, HALF:]


def _post_body_last(adda, addb, asda, asdb, adsa, adsb, assa, assb,
                    w0a, w0b, w1a, w1b, w2a, w2b, w3a, w3b,
                    b01, b23, gi_dd, gi_sd, gi_ds, gi_ss, alpha,
                    hd_o, hs_o):
    a = alpha[0, 0]
    dn = (_dot(adda[...], w0a[...]) + _dot(addb[...], w0b[...])) * _nrm(gi_dd[...])
    dn = dn + (_dot(asda[...], w1a[...]) + _dot(asdb[...], w1b[...])) * _nrm(gi_sd[...])
    dn = dn + b01[...]
    hd_o[...] = jnp.where(dn > 0, dn, a * dn)
    sn = (_dot(adsa[...], w2a[...]) + _dot(adsb[...], w2b[...])) * _nrm(gi_ds[...])
    sn = sn + (_dot(assa[...], w3a[...]) + _dot(assb[...], w3b[...])) * _nrm(gi_ss[...])
    sn = sn + b23[...]
    hs_o[...] = jnp.where(sn > 0, sn, a * sn)


def _tc_post(has_next, aggs, wslices, b01, b23, gins, alpha, gsrcs):
    nblk = N // _RB
    ins = list(aggs) + list(wslices) + [b01, b23] + list(gins) + [alpha]
    in_specs = ([_rows((_RB, HALF))] * 8 + [_full((HALF, D))] * 8
                + [_full((1, D))] * 2 + [_rows((_RB, 1))] * 4
                + [_full((1, 1))])
    if has_next:
        ins += list(gsrcs)
        in_specs += [_rows((_RB, 1))] * 4
        outs = ([jax.ShapeDtypeStruct((N, D), F32)] * 2
                + [jax.ShapeDtypeStruct((N, HALF), F32)] * 8)
        out_specs = [_rows((_RB, D))] * 2 + [_rows((_RB, HALF))] * 8
        body = _post_body_next
    else:
        outs = [jax.ShapeDtypeStruct((N, D), F32)] * 2
        out_specs = [_rows((_RB, D))] * 2
        body = _post_body_last
    return pl.pallas_call(
        body, grid=(nblk,), in_specs=in_specs, out_specs=out_specs,
        out_shape=outs,
    )(*ins)


def _res_body(hd0, hd1, hd2, hs0, hs1, hs2,
              wd0, wd1, wd2, ws0, ws1, ws2, bd, bs,
              fda, fdb, fsa, fsb):
    fd = (_dot(hd0[...], wd0[...]) + _dot(hd1[...], wd1[...])
          + _dot(hd2[...], wd2[...]) + bd[...])
    fs = (_dot(hs0[...], ws0[...]) + _dot(hs1[...], ws1[...])
          + _dot(hs2[...], ws2[...]) + bs[...])
    fda[...] = fd[:, :HALF]
    fdb[...] = fd[:, HALF:]
    fsa[...] = fs[:, :HALF]
    fsb[...] = fs[:, HALF:]


def _tc_res(hds, hss, wds, wss, bd, bs):
    nblk = N // _RB
    return pl.pallas_call(
        _res_body,
        grid=(nblk,),
        in_specs=[_rows((_RB, D))] * 6 + [_full((D, D))] * 6 + [_full((1, D))] * 2,
        out_specs=[_rows((_RB, HALF))] * 4,
        out_shape=[jax.ShapeDtypeStruct((N, HALF), F32)] * 4,
    )(*hds, *hss, *wds, *wss, bd, bs)


_MB = 512


def _m1_body(g0a, g0b, g1a, g1b, g2a, g2b, g3a, g3b,
             wdd_aa, wdd_ab, wdd_ba, wdd_bb,
             wdr_aa, wdr_ab, wdr_ba, wdr_bb,
             whd_a, whd_b, whs_a, whs_b,
             wm1_a, wm1_b, bm1, wm2, wins, wmlp_a, wmlp_b,
             sc_o, im_o, pr_o):
    f0a, f0b = g0a[...], g0b[...]
    f1a, f1b = g1a[...], g1b[...]
    f2a, f2b = g2a[...], g2b[...]
    f3a, f3b = g3a[...], g3b[...]
    p = (f0a + f1a) * 0.5
    q = (f0b + f1b) * 0.5
    dis_a = ((_dot(p, wdd_aa[...]) + _dot(q, wdd_ba[...]) + f2a) * 0.5 + f3a) * 0.5
    dis_b = ((_dot(p, wdd_ab[...]) + _dot(q, wdd_bb[...]) + f2b) * 0.5 + f3b) * 0.5
    p2 = (f3a + f2a) * 0.5
    q2 = (f3b + f2b) * 0.5
    drug_a = ((_dot(p2, wdr_aa[...]) + _dot(q2, wdr_ba[...]) + f1a) * 0.5 + f0a) * 0.5
    drug_b = ((_dot(p2, wdr_ab[...]) + _dot(q2, wdr_bb[...]) + f1b) * 0.5 + f0b) * 0.5
    di = _dot(drug_a, whd_a[...]) + _dot(drug_b, whd_b[...])
    si = _dot(dis_a, whs_a[...]) + _dot(dis_b, whs_b[...])
    t1 = jnp.tanh(_dot(di, wm1_a[...]) + _dot(si, wm1_b[...]) + bm1[...])
    sc_o[...] = _dot(t1, wm2[...])
    im_o[...] = _dot(di, wmlp_a[...]) + _dot(si, wmlp_b[...])
    pr_o[...] = jnp.sum(_dot(di, wins[...]) * si, axis=-1, keepdims=True)


def _tc_m1(gs, wq, wrq, whd, whs, wm1, bm1, wm2, wins, wmlp):
    nblk = (B * BAG) // _MB
    ins = list(gs) + list(wq) + list(wrq) + list(whd) + list(whs) + list(wm1) \
        + [bm1, wm2, wins] + list(wmlp)
    in_specs = ([_rows((_MB, HALF))] * 8
                + [_full((HALF, HALF))] * 8
                + [_full((HALF, HALF))] * 4
                + [_full((HALF, D))] * 2
                + [_full((1, D)), _full((D, 1)), _full((HALF, HALF))]
                + [_full((HALF, 1))] * 2)
    return pl.pallas_call(
        _m1_body, grid=(nblk,), in_specs=in_specs,
        out_specs=[_rows((_MB, 1))] * 3,
        out_shape=[jax.ShapeDtypeStruct((B * BAG, 1), F32)] * 3,
    )(*ins)


def _m2_body(sc_ref, im_ref, pr_ref, out_ref):
    scr = sc_ref[...]
    m = jnp.max(scr, axis=-1, keepdims=True)
    e = jnp.exp(scr - m)
    attn = e / jnp.sum(e, axis=-1, keepdims=True)
    mlp = jnp.sum(attn * im_ref[...], axis=-1, keepdims=True)
    ap = attn * pr_ref[...]
    iota = lax.broadcasted_iota(jnp.int32, ap.shape, 1)
    acc = jnp.zeros((ap.shape[0], 1), F32)
    cur = ap
    for _ in range(KTOP):
        mk = jnp.max(cur, axis=-1, keepdims=True)
        acc = acc + mk
        pos = jnp.min(jnp.where(cur == mk, iota, BAG), axis=-1, keepdims=True)
        cur = jnp.where(iota == pos, -3e38, cur)
    out_ref[...] = (mlp + acc * (1.0 / KTOP)) * 0.5


def _tc_m2(scores, imlp, pred):
    return pl.pallas_call(
        _m2_body, grid=(1,),
        in_specs=[_full((B, BAG))] * 3,
        out_specs=_full((B, 1)),
        out_shape=jax.ShapeDtypeStruct((B, 1), F32),
    )(scores, imlp, pred)


def _split_cols(w):
    return w[:HALF, :], w[HALF:, :]


def kernel(feat_drug, feat_disease, edge_index_drug_drug,
           edge_index_drug_disease, edge_index_disease_drug,
           edge_index_disease_disease, mp_ins,
           W_lin_drug, b_lin_drug, W_lin_dis, b_lin_dis, Wg, bg, a_prelu,
           W_res_drug, b_res_drug, W_res_dis, b_res_dis,
           W_dd, W_dr, W_half_drug, W_half_dis,
           W_mil1, b_mil1, W_mil2, W_ins, W_mlp):
    npad = EPAD - E
    pad_lo = (jnp.arange(npad, dtype=jnp.int32) % 16)
    pad_hi = pad_lo + N

    def pad_edges(ei):
        src = jnp.concatenate([ei[0], pad_lo]).reshape(NCH, CH)
        dst = jnp.concatenate([ei[1], pad_hi]).reshape(NCH, CH)
        srcd = jnp.concatenate([ei[0], pad_hi]).reshape(NCH, CH)
        dstd = jnp.concatenate([ei[1], pad_hi]).reshape(NCH, CH)
        return src, dst, srcd, dstd

    dd = pad_edges(edge_index_drug_drug)
    ds = pad_edges(edge_index_drug_disease)
    sd = pad_edges(edge_index_disease_drug)
    ss = pad_edges(edge_index_disease_disease)

    ones128 = jnp.ones((CH,), F32)
    zer640 = jnp.zeros((STRIP,), F32)
    zer_blk = jnp.zeros((CH, HALF), F32)

    degs = _sc_degrees(ones128, zer640,
                       dd[2], dd[3], ds[2], ds[3], sd[2], sd[3], ss[2], ss[3])
    (g_dd_s, g_dd_d, g_ds_s, g_ds_d,
     g_sd_s, g_sd_d, g_ss_s, g_ss_d) = [g[:N].reshape(N, 1) for g in degs]

    bd = b_lin_drug.reshape(1, D)
    bs = b_lin_dis.reshape(1, D)
    pre = _tc_pre(feat_drug, feat_disease, W_lin_drug, W_lin_dis, bd, bs,
                  g_dd_s, g_ds_s, g_sd_s, g_ss_s)
    hd0, hs0 = pre[0], pre[1]
    tables = pre[2:]

    hs_list = [hd0]
    hss_list = [hs0]
    hd_cur, hs_cur = hd0, hs0
    gins = (g_dd_d, g_sd_d, g_ds_d, g_ss_d)
    gsrcs = (g_dd_s, g_ds_s, g_sd_s, g_ss_s)
    for layer in range(2):
        xdda, xddb, xdsa, xdsb, xsda, xsdb, xssa, xssb = tables
        aggs8 = _sc_agg(zer_blk,
                        dd[0], dd[1], sd[0], sd[1], ds[0], ds[1], ss[0], ss[1],
                        xdda, xddb, xsda, xsdb, xdsa, xdsb, xssa, xssb)
        agg = [a[:N, :] for a in aggs8]
        wsl = []
        for r in range(4):
            wa, wb = _split_cols(Wg[layer, r])
            wsl += [wa, wb]
        b01 = (bg[layer, 0] + bg[layer, 1]).reshape(1, D)
        b23 = (bg[layer, 2] + bg[layer, 3]).reshape(1, D)
        alpha = a_prelu[layer].reshape(1, 1)
        has_next = layer + 1 < 2
        post = _tc_post(has_next, agg, wsl, b01, b23, gins, alpha, gsrcs)
        hd_cur, hs_cur = post[0], post[1]
        hs_list.append(hd_cur)
        hss_list.append(hs_cur)
        if has_next:
            tables = post[2:]

    wd_sl = [W_res_drug[i * D:(i + 1) * D, :] for i in range(3)]
    ws_sl = [W_res_dis[i * D:(i + 1) * D, :] for i in range(3)]
    fda, fdb, fsa, fsb = _tc_res(hs_list, hss_list, wd_sl, ws_sl,
                                 b_res_drug.reshape(1, D),
                                 b_res_dis.reshape(1, D))

    idx = [mp_ins[:, :, j].reshape((B * BAG) // CH, CH) for j in range(4)]
    gs = _sc_meta(idx[0], idx[1], idx[2], idx[3], fda, fdb, fsa, fsb)

    wq = [W_dd[:HALF, :HALF], W_dd[:HALF, HALF:],
          W_dd[HALF:, :HALF], W_dd[HALF:, HALF:]]
    wrq = [W_dr[:HALF, :HALF], W_dr[:HALF, HALF:],
           W_dr[HALF:, :HALF], W_dr[HALF:, HALF:]]
    whd = _split_cols(W_half_drug)
    whs = _split_cols(W_half_dis)
    wm1 = _split_cols(W_mil1)
    wmlp = _split_cols(W_mlp)
    scores, imlp, pred = _tc_m1(gs, wq, wrq, whd, whs, wm1,
                                b_mil1.reshape(1, D), W_mil2, W_ins, wmlp)
    return _tc_m2(scores.reshape(B, BAG), imlp.reshape(B, BAG),
                  pred.reshape(B, BAG))


# speedup vs baseline: 5.8570x; 1.4391x over previous
import functools

import jax
import jax.numpy as jnp
from jax import lax
from jax.experimental import pallas as pl
from jax.experimental.pallas import tpu as pltpu
from jax.experimental.pallas import tpu_sc as plsc

N = 10000
D = 256
HALF = 128
E = 320000
CH = 128
NCH = 2560
EPAD = NCH * CH
APAD = 10240
STRIP = APAD // 16
B = 1024
BAG = 64
KTOP = 5
NS = 16
NC = 2
TPC = NCH // NS
MCH = (B * BAG) // CH // (NC * NS)

F32 = jnp.float32
_SC_CACHE = {}


def _sc_mesh():
    return plsc.VectorSubcoreMesh(core_axis_name="c", subcore_axis_name="s")


def _sc_degrees(*args):
    if "deg" not in _SC_CACHE:
        _SC_CACHE["deg"] = functools.partial(
            pl.kernel,
            mesh=_sc_mesh(),
            out_type=[jax.ShapeDtypeStruct((APAD,), F32) for _ in range(8)],
            scratch_types=[
                pltpu.VMEM((TPC, CH), jnp.int32),
                pltpu.VMEM((CH,), F32),
                pltpu.VMEM((STRIP,), F32),
                pltpu.VMEM_SHARED((APAD,), F32),
                pltpu.VMEM_SHARED((APAD,), F32),
                pltpu.VMEM_SHARED((APAD,), F32),
                pltpu.VMEM_SHARED((APAD,), F32),
            ],
        )(_sc_degrees_body)
    return _SC_CACHE["deg"](*args)


def _sc_degrees_body(ones_hbm, zeros_hbm,
                dd_s, dd_d, ds_s, ds_d, sd_s, sd_d, ss_s, ss_d,
                o0, o1, o2, o3, o4, o5, o6, o7,
                ibuf, ones_v, zer_v, h0, h1, h2, h3):
    c = lax.axis_index("c")
    s = lax.axis_index("s")
    pltpu.sync_copy(ones_hbm, ones_v)
    pltpu.sync_copy(zeros_hbm, zer_v)
    hists = [h0, h1, h2, h3]
    for h in hists:
        pltpu.sync_copy(zer_v, h.at[pl.ds(s * STRIP, STRIP)])
    plsc.subcore_barrier()

    def hist_pass(idx_hbm, hist):
        pltpu.sync_copy(idx_hbm.at[pl.ds(s * TPC, TPC)], ibuf)

        def body(j, carry):
            pltpu.sync_copy(ones_v, hist.at[ibuf.at[j]], add=True)
            return carry

        lax.fori_loop(0, TPC, body, 0)

    @pl.when(c == 0)
    def _():
        for idx, h in zip([dd_s, dd_d, ds_s, ds_d], hists):
            hist_pass(idx, h)

    @pl.when(c == 1)
    def _():
        for idx, h in zip([sd_s, sd_d, ss_s, ss_d], hists):
            hist_pass(idx, h)

    plsc.subcore_barrier()
    outs01 = [o0, o1, o2, o3]
    outs23 = [o4, o5, o6, o7]

    @pl.when(c == 0)
    def _():
        for h, o in zip(hists, outs01):
            pltpu.sync_copy(h.at[pl.ds(s * STRIP, STRIP)],
                            o.at[pl.ds(s * STRIP, STRIP)])

    @pl.when(c == 1)
    def _():
        for h, o in zip(hists, outs23):
            pltpu.sync_copy(h.at[pl.ds(s * STRIP, STRIP)],
                            o.at[pl.ds(s * STRIP, STRIP)])


def _sc_agg(*args):
    if "agg" not in _SC_CACHE:
        _SC_CACHE["agg"] = functools.partial(
            pl.kernel,
            mesh=_sc_mesh(),
            out_type=[jax.ShapeDtypeStruct((APAD, HALF), F32) for _ in range(8)],
            scratch_types=[
                pltpu.VMEM((TPC // 4, CH), jnp.int32),
                pltpu.VMEM((TPC // 4, CH), jnp.int32),
                pltpu.VMEM((CH, HALF), F32),
                pltpu.VMEM((CH, HALF), F32),
                pltpu.SemaphoreType.DMA,
                pltpu.SemaphoreType.DMA,
                pltpu.VMEM_SHARED((APAD, HALF), F32),
            ],
        )(_sc_agg_body)
    return _SC_CACHE["agg"](*args)


def _sc_agg_body(zeros_hbm,
            src0, dst0, src1, dst1, src2, dst2, src3, dst3,
            tA0, tB0, tA1, tB1, tA2, tB2, tA3, tB3,
            oA0, oB0, oA1, oB1, oA2, oB2, oA3, oB3,
            sbuf, dbuf, u0, u1, sem0, sem1, acc):
    c = lax.axis_index("c")
    s = lax.axis_index("s")
    edges = [(src0, dst0), (src1, dst1), (src2, dst2), (src3, dst3)]
    tabs = [(tA0, tB0), (tA1, tB1), (tA2, tB2), (tA3, tB3)]
    outs = [(oA0, oB0), (oA1, oB1), (oA2, oB2), (oA3, oB3)]
    qc = TPC // 4
    for r in range(4):
        pltpu.sync_copy(zeros_hbm, u0)
        for k2 in range(STRIP // CH):
            pltpu.sync_copy(u0, acc.at[pl.ds(s * STRIP + k2 * CH, CH)])
        plsc.subcore_barrier()
        srcp, dstp = edges[r]

        def process(tbl, srcp=srcp, dstp=dstp):
            for hv in range(4):
                pltpu.sync_copy(srcp.at[pl.ds(s * TPC + hv * qc, qc)], sbuf)
                pltpu.sync_copy(dstp.at[pl.ds(s * TPC + hv * qc, qc)], dbuf)
                pltpu.async_copy(tbl.at[sbuf.at[0]], u0, sem0)

                def body(jj, carry):
                    pltpu.async_copy(tbl.at[sbuf.at[jj * 2 + 1]], u1, sem1)
                    pltpu.make_async_copy(tbl.at[sbuf.at[0]], u0, sem0).wait()
                    pltpu.sync_copy(u0, acc.at[dbuf.at[jj * 2]], add=True)
                    nxt = jnp.minimum(jj * 2 + 2, qc - 1)
                    pltpu.async_copy(tbl.at[sbuf.at[nxt]], u0, sem0)
                    pltpu.make_async_copy(tbl.at[sbuf.at[0]], u1, sem1).wait()
                    pltpu.sync_copy(u1, acc.at[dbuf.at[jj * 2 + 1]], add=True)
                    return carry

                lax.fori_loop(0, qc // 2, body, 0)
                pltpu.make_async_copy(tbl.at[sbuf.at[0]], u0, sem0).wait()

        tA, tB = tabs[r]

        @pl.when(c == 0)
        def _():
            process(tA)

        @pl.when(c == 1)
        def _():
            process(tB)

        plsc.subcore_barrier()
        oA, oB = outs[r]

        @pl.when(c == 0)
        def _():
            pltpu.sync_copy(acc.at[pl.ds(s * STRIP, STRIP)],
                            oA.at[pl.ds(s * STRIP, STRIP)])

        @pl.when(c == 1)
        def _():
            pltpu.sync_copy(acc.at[pl.ds(s * STRIP, STRIP)],
                            oB.at[pl.ds(s * STRIP, STRIP)])


def _sc_meta(*args):
    if "meta" not in _SC_CACHE:
        _SC_CACHE["meta"] = functools.partial(
            pl.kernel,
            mesh=_sc_mesh(),
            out_type=[jax.ShapeDtypeStruct((B * BAG, HALF), F32) for _ in range(8)],
            scratch_types=[
                pltpu.VMEM((MCH, CH), jnp.int32),
                pltpu.VMEM((CH, HALF), F32),
                pltpu.VMEM((CH, HALF), F32),
                pltpu.SemaphoreType.DMA,
                pltpu.SemaphoreType.DMA,
            ],
        )(_sc_meta_body)
    return _SC_CACHE["meta"](*args)


def _sc_meta_body(idx0, idx1, idx2, idx3, fdA, fdB, fsA, fsB,
             g0A, g0B, g1A, g1B, g2A, g2B, g3A, g3B,
             ibuf, u0, u1, sem0, sem1):
    c = lax.axis_index("c")
    s = lax.axis_index("s")
    wid = s * NC + c
    jobs = [(idx0, fdA, g0A), (idx0, fdB, g0B),
            (idx1, fdA, g1A), (idx1, fdB, g1B),
            (idx2, fsA, g2A), (idx2, fsB, g2B),
            (idx3, fsA, g3A), (idx3, fsB, g3B)]
    for idx, tbl, out in jobs:
        pltpu.sync_copy(idx.at[pl.ds(wid * MCH, MCH)], ibuf)
        pltpu.async_copy(tbl.at[ibuf.at[0]], u0, sem0)

        def body(jj, carry, tbl=tbl, out=out):
            pltpu.async_copy(tbl.at[ibuf.at[jj * 2 + 1]], u1, sem1)
            pltpu.make_async_copy(tbl.at[ibuf.at[0]], u0, sem0).wait()
            pltpu.sync_copy(u0, out.at[pl.ds((wid * MCH + jj * 2) * CH, CH)])
            nxt = jnp.minimum(jj * 2 + 2, MCH - 1)
            pltpu.async_copy(tbl.at[ibuf.at[nxt]], u0, sem0)
            pltpu.make_async_copy(tbl.at[ibuf.at[0]], u1, sem1).wait()
            pltpu.sync_copy(u1, out.at[pl.ds((wid * MCH + jj * 2 + 1) * CH, CH)])
            return carry

        lax.fori_loop(0, MCH // 2, body, 0)
        pltpu.make_async_copy(tbl.at[ibuf.at[0]], u0, sem0).wait()


_RB = 1000


def _dot(a, b):
    return jnp.dot(a, b, preferred_element_type=F32)


def _full(shape):
    return pl.BlockSpec(shape, lambda i: (0,) * len(shape))


def _rows(shape):
    return pl.BlockSpec(shape, lambda i: (i,) + (0,) * (len(shape) - 1))


def _nrm(deg):
    return lax.rsqrt(jnp.maximum(deg, 1.0))


def _pre_body(fd, fs, wd, ws, bd, bs, g_dd, g_ds, g_sd, g_ss,
              hd_o, hs_o, xdda, xddb, xdsa, xdsb, xsda, xsdb, xssa, xssb):
    hd = _dot(fd[...], wd[...]) + bd[...]
    hs = _dot(fs[...], ws[...]) + bs[...]
    hd_o[...] = hd
    hs_o[...] = hs
    xdd = hd * _nrm(g_dd[...])
    xds = hd * _nrm(g_ds[...])
    xsd = hs * _nrm(g_sd[...])
    xss = hs * _nrm(g_ss[...])
    xdda[...] = xdd[:, :HALF]
    xddb[...] = xdd[:, HALF:]
    xdsa[...] = xds[:, :HALF]
    xdsb[...] = xds[:, HALF:]
    xsda[...] = xsd[:, :HALF]
    xsdb[...] = xsd[:, HALF:]
    xssa[...] = xss[:, :HALF]
    xssb[...] = xss[:, HALF:]


def _tc_pre(fd, fs, wd, ws, bd, bs, g_dd, g_ds, g_sd, g_ss):
    nblk = N // _RB
    outs = ([jax.ShapeDtypeStruct((N, D), F32)] * 2
            + [jax.ShapeDtypeStruct((N, HALF), F32)] * 8)
    return pl.pallas_call(
        _pre_body,
        grid=(nblk,),
        in_specs=[_rows((_RB, D)), _rows((_RB, D)),
                  _full((D, D)), _full((D, D)),
                  _full((1, D)), _full((1, D)),
                  _rows((_RB, 1)), _rows((_RB, 1)),
                  _rows((_RB, 1)), _rows((_RB, 1))],
        out_specs=[_rows((_RB, D))] * 2 + [_rows((_RB, HALF))] * 8,
        out_shape=outs,
    )(fd, fs, wd, ws, bd, bs, g_dd, g_ds, g_sd, g_ss)


def _post_body_next(adda, addb, asda, asdb, adsa, adsb, assa, assb,
                    w0a, w0b, w1a, w1b, w2a, w2b, w3a, w3b,
                    b01, b23, gi_dd, gi_sd, gi_ds, gi_ss, alpha,
                    g_dd, g_ds, g_sd, g_ss,
                    hd_o, hs_o, xdda, xddb, xdsa, xdsb, xsda, xsdb, xssa, xssb):
    a = alpha[0, 0]
    dn = (_dot(adda[...], w0a[...]) + _dot(addb[...], w0b[...])) * _nrm(gi_dd[...])
    dn = dn + (_dot(asda[...], w1a[...]) + _dot(asdb[...], w1b[...])) * _nrm(gi_sd[...])
    dn = dn + b01[...]
    hd = jnp.where(dn > 0, dn, a * dn)
    sn = (_dot(adsa[...], w2a[...]) + _dot(adsb[...], w2b[...])) * _nrm(gi_ds[...])
    sn = sn + (_dot(assa[...], w3a[...]) + _dot(assb[...], w3b[...])) * _nrm(gi_ss[...])
    sn = sn + b23[...]
    hs = jnp.where(sn > 0, sn, a * sn)
    hd_o[...] = hd
    hs_o[...] = hs
    xdd = hd * _nrm(g_dd[...])
    xds = hd * _nrm(g_ds[...])
    xsd = hs * _nrm(g_sd[...])
    xss = hs * _nrm(g_ss[...])
    xdda[...] = xdd[:, :HALF]
    xddb[...] = xdd[:, HALF:]
    xdsa[...] = xds[:, :HALF]
    xdsb[...] = xds[:, HALF:]
    xsda[...] = xsd[:, :HALF]
    xsdb[...] = xsd[:, HALF:]
    xssa[...] = xss[:, :HALF]
    xssb[...] = xss[:, HALF:]


def _post_body_last(adda, addb, asda, asdb, adsa, adsb, assa, assb,
                    w0a, w0b, w1a, w1b, w2a, w2b, w3a, w3b,
                    b01, b23, gi_dd, gi_sd, gi_ds, gi_ss, alpha,
                    hd_o, hs_o):
    a = alpha[0, 0]
    dn = (_dot(adda[...], w0a[...]) + _dot(addb[...], w0b[...])) * _nrm(gi_dd[...])
    dn = dn + (_dot(asda[...], w1a[...]) + _dot(asdb[...], w1b[...])) * _nrm(gi_sd[...])
    dn = dn + b01[...]
    hd_o[...] = jnp.where(dn > 0, dn, a * dn)
    sn = (_dot(adsa[...], w2a[...]) + _dot(adsb[...], w2b[...])) * _nrm(gi_ds[...])
    sn = sn + (_dot(assa[...], w3a[...]) + _dot(assb[...], w3b[...])) * _nrm(gi_ss[...])
    sn = sn + b23[...]
    hs_o[...] = jnp.where(sn > 0, sn, a * sn)


def _tc_post(has_next, aggs, wslices, b01, b23, gins, alpha, gsrcs):
    nblk = N // _RB
    ins = list(aggs) + list(wslices) + [b01, b23] + list(gins) + [alpha]
    in_specs = ([_rows((_RB, HALF))] * 8 + [_full((HALF, D))] * 8
                + [_full((1, D))] * 2 + [_rows((_RB, 1))] * 4
                + [_full((1, 1))])
    if has_next:
        ins += list(gsrcs)
        in_specs += [_rows((_RB, 1))] * 4
        outs = ([jax.ShapeDtypeStruct((N, D), F32)] * 2
                + [jax.ShapeDtypeStruct((N, HALF), F32)] * 8)
        out_specs = [_rows((_RB, D))] * 2 + [_rows((_RB, HALF))] * 8
        body = _post_body_next
    else:
        outs = [jax.ShapeDtypeStruct((N, D), F32)] * 2
        out_specs = [_rows((_RB, D))] * 2
        body = _post_body_last
    return pl.pallas_call(
        body, grid=(nblk,), in_specs=in_specs, out_specs=out_specs,
        out_shape=outs,
    )(*ins)


def _res_body(hd0, hd1, hd2, hs0, hs1, hs2,
              wd0, wd1, wd2, ws0, ws1, ws2, bd, bs,
              fda, fdb, fsa, fsb):
    fd = (_dot(hd0[...], wd0[...]) + _dot(hd1[...], wd1[...])
          + _dot(hd2[...], wd2[...]) + bd[...])
    fs = (_dot(hs0[...], ws0[...]) + _dot(hs1[...], ws1[...])
          + _dot(hs2[...], ws2[...]) + bs[...])
    fda[...] = fd[:, :HALF]
    fdb[...] = fd[:, HALF:]
    fsa[...] = fs[:, :HALF]
    fsb[...] = fs[:, HALF:]


def _tc_res(hds, hss, wds, wss, bd, bs):
    nblk = N // _RB
    return pl.pallas_call(
        _res_body,
        grid=(nblk,),
        in_specs=[_rows((_RB, D))] * 6 + [_full((D, D))] * 6 + [_full((1, D))] * 2,
        out_specs=[_rows((_RB, HALF))] * 4,
        out_shape=[jax.ShapeDtypeStruct((N, HALF), F32)] * 4,
    )(*hds, *hss, *wds, *wss, bd, bs)


_MB = 512


def _m1_body(g0a, g0b, g1a, g1b, g2a, g2b, g3a, g3b,
             wdd_aa, wdd_ab, wdd_ba, wdd_bb,
             wdr_aa, wdr_ab, wdr_ba, wdr_bb,
             whd_a, whd_b, whs_a, whs_b,
             wm1_a, wm1_b, bm1, wm2, wins, wmlp_a, wmlp_b,
             sc_o, im_o, pr_o):
    f0a, f0b = g0a[...], g0b[...]
    f1a, f1b = g1a[...], g1b[...]
    f2a, f2b = g2a[...], g2b[...]
    f3a, f3b = g3a[...], g3b[...]
    p = (f0a + f1a) * 0.5
    q = (f0b + f1b) * 0.5
    dis_a = ((_dot(p, wdd_aa[...]) + _dot(q, wdd_ba[...]) + f2a) * 0.5 + f3a) * 0.5
    dis_b = ((_dot(p, wdd_ab[...]) + _dot(q, wdd_bb[...]) + f2b) * 0.5 + f3b) * 0.5
    p2 = (f3a + f2a) * 0.5
    q2 = (f3b + f2b) * 0.5
    drug_a = ((_dot(p2, wdr_aa[...]) + _dot(q2, wdr_ba[...]) + f1a) * 0.5 + f0a) * 0.5
    drug_b = ((_dot(p2, wdr_ab[...]) + _dot(q2, wdr_bb[...]) + f1b) * 0.5 + f0b) * 0.5
    di = _dot(drug_a, whd_a[...]) + _dot(drug_b, whd_b[...])
    si = _dot(dis_a, whs_a[...]) + _dot(dis_b, whs_b[...])
    t1 = jnp.tanh(_dot(di, wm1_a[...]) + _dot(si, wm1_b[...]) + bm1[...])
    sc_o[...] = _dot(t1, wm2[...])
    im_o[...] = _dot(di, wmlp_a[...]) + _dot(si, wmlp_b[...])
    pr_o[...] = jnp.sum(_dot(di, wins[...]) * si, axis=-1, keepdims=True)


def _tc_m1(gs, wq, wrq, whd, whs, wm1, bm1, wm2, wins, wmlp):
    nblk = (B * BAG) // _MB
    ins = list(gs) + list(wq) + list(wrq) + list(whd) + list(whs) + list(wm1) \
        + [bm1, wm2, wins] + list(wmlp)
    in_specs = ([_rows((_MB, HALF))] * 8
                + [_full((HALF, HALF))] * 8
                + [_full((HALF, HALF))] * 4
                + [_full((HALF, D))] * 2
                + [_full((1, D)), _full((D, 1)), _full((HALF, HALF))]
                + [_full((HALF, 1))] * 2)
    return pl.pallas_call(
        _m1_body, grid=(nblk,), in_specs=in_specs,
        out_specs=[_rows((_MB, 1))] * 3,
        out_shape=[jax.ShapeDtypeStruct((B * BAG, 1), F32)] * 3,
    )(*ins)


def _m2_body(sc_ref, im_ref, pr_ref, out_ref):
    scr = sc_ref[...]
    m = jnp.max(scr, axis=-1, keepdims=True)
    e = jnp.exp(scr - m)
    attn = e / jnp.sum(e, axis=-1, keepdims=True)
    mlp = jnp.sum(attn * im_ref[...], axis=-1, keepdims=True)
    ap = attn * pr_ref[...]
    iota = lax.broadcasted_iota(jnp.int32, ap.shape, 1)
    acc = jnp.zeros((ap.shape[0], 1), F32)
    cur = ap
    for _ in range(KTOP):
        mk = jnp.max(cur, axis=-1, keepdims=True)
        acc = acc + mk
        pos = jnp.min(jnp.where(cur == mk, iota, BAG), axis=-1, keepdims=True)
        cur = jnp.where(iota == pos, -3e38, cur)
    out_ref[...] = (mlp + acc * (1.0 / KTOP)) * 0.5


def _tc_m2(scores, imlp, pred):
    return pl.pallas_call(
        _m2_body, grid=(1,),
        in_specs=[_full((B, BAG))] * 3,
        out_specs=_full((B, 1)),
        out_shape=jax.ShapeDtypeStruct((B, 1), F32),
    )(scores, imlp, pred)


def _split_cols(w):
    return w[:HALF, :], w[HALF:, :]


def kernel(feat_drug, feat_disease, edge_index_drug_drug,
           edge_index_drug_disease, edge_index_disease_drug,
           edge_index_disease_disease, mp_ins,
           W_lin_drug, b_lin_drug, W_lin_dis, b_lin_dis, Wg, bg, a_prelu,
           W_res_drug, b_res_drug, W_res_dis, b_res_dis,
           W_dd, W_dr, W_half_drug, W_half_dis,
           W_mil1, b_mil1, W_mil2, W_ins, W_mlp):
    npad = EPAD - E
    pad_lo = (jnp.arange(npad, dtype=jnp.int32) % 16)
    pad_hi = pad_lo + N

    def pad_edges(ei):
        src = jnp.concatenate([ei[0], pad_lo]).reshape(NCH, CH)
        dst = jnp.concatenate([ei[1], pad_hi]).reshape(NCH, CH)
        srcd = jnp.concatenate([ei[0], pad_hi]).reshape(NCH, CH)
        dstd = jnp.concatenate([ei[1], pad_hi]).reshape(NCH, CH)
        return src, dst, srcd, dstd

    dd = pad_edges(edge_index_drug_drug)
    ds = pad_edges(edge_index_drug_disease)
    sd = pad_edges(edge_index_disease_drug)
    ss = pad_edges(edge_index_disease_disease)

    ones128 = jnp.ones((CH,), F32)
    zer640 = jnp.zeros((STRIP,), F32)
    zer_blk = jnp.zeros((CH, HALF), F32)

    degs = _sc_degrees(ones128, zer640,
                       dd[2], dd[3], ds[2], ds[3], sd[2], sd[3], ss[2], ss[3])
    (g_dd_s, g_dd_d, g_ds_s, g_ds_d,
     g_sd_s, g_sd_d, g_ss_s, g_ss_d) = [g[:N].reshape(N, 1) for g in degs]

    bd = b_lin_drug.reshape(1, D)
    bs = b_lin_dis.reshape(1, D)
    pre = _tc_pre(feat_drug, feat_disease, W_lin_drug, W_lin_dis, bd, bs,
                  g_dd_s, g_ds_s, g_sd_s, g_ss_s)
    hd0, hs0 = pre[0], pre[1]
    tables = pre[2:]

    hs_list = [hd0]
    hss_list = [hs0]
    hd_cur, hs_cur = hd0, hs0
    gins = (g_dd_d, g_sd_d, g_ds_d, g_ss_d)
    gsrcs = (g_dd_s, g_ds_s, g_sd_s, g_ss_s)
    for layer in range(2):
        xdda, xddb, xdsa, xdsb, xsda, xsdb, xssa, xssb = tables
        aggs8 = _sc_agg(zer_blk,
                        dd[0], dd[1], sd[0], sd[1], ds[0], ds[1], ss[0], ss[1],
                        xdda, xddb, xsda, xsdb, xdsa, xdsb, xssa, xssb)
        agg = [a[:N, :] for a in aggs8]
        wsl = []
        for r in range(4):
            wa, wb = _split_cols(Wg[layer, r])
            wsl += [wa, wb]
        b01 = (bg[layer, 0] + bg[layer, 1]).reshape(1, D)
        b23 = (bg[layer, 2] + bg[layer, 3]).reshape(1, D)
        alpha = a_prelu[layer].reshape(1, 1)
        has_next = layer + 1 < 2
        post = _tc_post(has_next, agg, wsl, b01, b23, gins, alpha, gsrcs)
        hd_cur, hs_cur = post[0], post[1]
        hs_list.append(hd_cur)
        hss_list.append(hs_cur)
        if has_next:
            tables = post[2:]

    wd_sl = [W_res_drug[i * D:(i + 1) * D, :] for i in range(3)]
    ws_sl = [W_res_dis[i * D:(i + 1) * D, :] for i in range(3)]
    fda, fdb, fsa, fsb = _tc_res(hs_list, hss_list, wd_sl, ws_sl,
                                 b_res_drug.reshape(1, D),
                                 b_res_dis.reshape(1, D))

    idx = [mp_ins[:, :, j].reshape((B * BAG) // CH, CH) for j in range(4)]
    gs = _sc_meta(idx[0], idx[1], idx[2], idx[3], fda, fdb, fsa, fsb)

    wq = [W_dd[:HALF, :HALF], W_dd[:HALF, HALF:],
          W_dd[HALF:, :HALF], W_dd[HALF:, HALF:]]
    wrq = [W_dr[:HALF, :HALF], W_dr[:HALF, HALF:],
           W_dr[HALF:, :HALF], W_dr[HALF:, HALF:]]
    whd = _split_cols(W_half_drug)
    whs = _split_cols(W_half_dis)
    wm1 = _split_cols(W_mil1)
    wmlp = _split_cols(W_mlp)
    scores, imlp, pred = _tc_m1(gs, wq, wrq, whd, whs, wm1,
                                b_mil1.reshape(1, D), W_mil2, W_ins, wmlp)
    return _tc_m2(scores.reshape(B, BAG), imlp.reshape(B, BAG),
                  pred.reshape(B, BAG))


# speedup vs baseline: 5.9085x; 1.0088x over previous
import functools

import jax
import jax.numpy as jnp
from jax import lax
from jax.experimental import pallas as pl
from jax.experimental.pallas import tpu as pltpu
from jax.experimental.pallas import tpu_sc as plsc

N = 10000
D = 256
HALF = 128
E = 320000
CH = 128
NCH = 2560
EPAD = NCH * CH
APAD = 10240
STRIP = APAD // 16
B = 1024
BAG = 64
KTOP = 5
NS = 16
NC = 2
TPC = NCH // NS
MCH = (B * BAG) // CH // (NC * NS)

F32 = jnp.float32
_SC_CACHE = {}


def _sc_mesh():
    return plsc.VectorSubcoreMesh(core_axis_name="c", subcore_axis_name="s")


def _sc_degrees(*args):
    if "deg" not in _SC_CACHE:
        _SC_CACHE["deg"] = functools.partial(
            pl.kernel,
            mesh=_sc_mesh(),
            out_type=[jax.ShapeDtypeStruct((APAD,), F32) for _ in range(8)],
            scratch_types=[
                pltpu.VMEM((TPC, CH), jnp.int32),
                pltpu.VMEM((CH,), F32),
                pltpu.VMEM((STRIP,), F32),
                pltpu.VMEM_SHARED((APAD,), F32),
                pltpu.VMEM_SHARED((APAD,), F32),
                pltpu.VMEM_SHARED((APAD,), F32),
                pltpu.VMEM_SHARED((APAD,), F32),
            ],
        )(_sc_degrees_body)
    return _SC_CACHE["deg"](*args)


def _sc_degrees_body(ones_hbm, zeros_hbm,
                dd_s, dd_d, ds_s, ds_d, sd_s, sd_d, ss_s, ss_d,
                o0, o1, o2, o3, o4, o5, o6, o7,
                ibuf, ones_v, zer_v, h0, h1, h2, h3):
    c = lax.axis_index("c")
    s = lax.axis_index("s")
    pltpu.sync_copy(ones_hbm, ones_v)
    pltpu.sync_copy(zeros_hbm, zer_v)
    hists = [h0, h1, h2, h3]
    for h in hists:
        pltpu.sync_copy(zer_v, h.at[pl.ds(s * STRIP, STRIP)])
    plsc.subcore_barrier()

    def hist_pass(idx_hbm, hist):
        pltpu.sync_copy(idx_hbm.at[pl.ds(s * TPC, TPC)], ibuf)

        def body(j, carry):
            pltpu.sync_copy(ones_v, hist.at[ibuf.at[j]], add=True)
            return carry

        lax.fori_loop(0, TPC, body, 0)

    @pl.when(c == 0)
    def _():
        for idx, h in zip([dd_s, dd_d, ds_s, ds_d], hists):
            hist_pass(idx, h)

    @pl.when(c == 1)
    def _():
        for idx, h in zip([sd_s, sd_d, ss_s, ss_d], hists):
            hist_pass(idx, h)

    plsc.subcore_barrier()
    outs01 = [o0, o1, o2, o3]
    outs23 = [o4, o5, o6, o7]

    @pl.when(c == 0)
    def _():
        for h, o in zip(hists, outs01):
            pltpu.sync_copy(h.at[pl.ds(s * STRIP, STRIP)],
                            o.at[pl.ds(s * STRIP, STRIP)])

    @pl.when(c == 1)
    def _():
        for h, o in zip(hists, outs23):
            pltpu.sync_copy(h.at[pl.ds(s * STRIP, STRIP)],
                            o.at[pl.ds(s * STRIP, STRIP)])


def _sc_agg(*args):
    if "agg" not in _SC_CACHE:
        _SC_CACHE["agg"] = functools.partial(
            pl.kernel,
            mesh=_sc_mesh(),
            out_type=[jax.ShapeDtypeStruct((APAD, HALF), F32) for _ in range(8)],
            scratch_types=[
                pltpu.VMEM((TPC // 4, CH), jnp.int32),
                pltpu.VMEM((TPC // 4, CH), jnp.int32),
                pltpu.VMEM((CH, HALF), F32),
                pltpu.VMEM((CH, HALF), F32),
                pltpu.SemaphoreType.DMA,
                pltpu.SemaphoreType.DMA,
                pltpu.VMEM_SHARED((APAD, HALF), F32),
            ],
        )(_sc_agg_body)
    return _SC_CACHE["agg"](*args)


def _sc_agg_body(zeros_hbm,
            src0, dst0, src1, dst1, src2, dst2, src3, dst3,
            tA0, tB0, tA1, tB1, tA2, tB2, tA3, tB3,
            oA0, oB0, oA1, oB1, oA2, oB2, oA3, oB3,
            sbuf, dbuf, u0, u1, sem0, sem1, acc):
    c = lax.axis_index("c")
    s = lax.axis_index("s")
    edges = [(src0, dst0), (src1, dst1), (src2, dst2), (src3, dst3)]
    tabs = [(tA0, tB0), (tA1, tB1), (tA2, tB2), (tA3, tB3)]
    outs = [(oA0, oB0), (oA1, oB1), (oA2, oB2), (oA3, oB3)]
    qc = TPC // 4
    for r in range(4):
        pltpu.sync_copy(zeros_hbm, acc.at[pl.ds(s * STRIP, STRIP)])
        plsc.subcore_barrier()
        srcp, dstp = edges[r]

        def process(tbl, srcp=srcp, dstp=dstp):
            for hv in range(4):
                pltpu.sync_copy(srcp.at[pl.ds(s * TPC + hv * qc, qc)], sbuf)
                pltpu.sync_copy(dstp.at[pl.ds(s * TPC + hv * qc, qc)], dbuf)
                pltpu.async_copy(tbl.at[sbuf.at[0]], u0, sem0)

                def body(jj, carry):
                    pltpu.async_copy(tbl.at[sbuf.at[jj * 2 + 1]], u1, sem1)
                    pltpu.make_async_copy(tbl.at[sbuf.at[0]], u0, sem0).wait()
                    pltpu.sync_copy(u0, acc.at[dbuf.at[jj * 2]], add=True)
                    nxt = jnp.minimum(jj * 2 + 2, qc - 1)
                    pltpu.async_copy(tbl.at[sbuf.at[nxt]], u0, sem0)
                    pltpu.make_async_copy(tbl.at[sbuf.at[0]], u1, sem1).wait()
                    pltpu.sync_copy(u1, acc.at[dbuf.at[jj * 2 + 1]], add=True)
                    return carry

                lax.fori_loop(0, qc // 2, body, 0)
                pltpu.make_async_copy(tbl.at[sbuf.at[0]], u0, sem0).wait()

        tA, tB = tabs[r]

        @pl.when(c == 0)
        def _():
            process(tA)

        @pl.when(c == 1)
        def _():
            process(tB)

        plsc.subcore_barrier()
        oA, oB = outs[r]

        @pl.when(c == 0)
        def _():
            pltpu.sync_copy(acc.at[pl.ds(s * STRIP, STRIP)],
                            oA.at[pl.ds(s * STRIP, STRIP)])

        @pl.when(c == 1)
        def _():
            pltpu.sync_copy(acc.at[pl.ds(s * STRIP, STRIP)],
                            oB.at[pl.ds(s * STRIP, STRIP)])


def _sc_meta(*args):
    if "meta" not in _SC_CACHE:
        _SC_CACHE["meta"] = functools.partial(
            pl.kernel,
            mesh=_sc_mesh(),
            out_type=[jax.ShapeDtypeStruct((B * BAG, HALF), F32) for _ in range(8)],
            scratch_types=[
                pltpu.VMEM((MCH, CH), jnp.int32),
                pltpu.VMEM((CH, HALF), F32),
                pltpu.VMEM((CH, HALF), F32),
                pltpu.SemaphoreType.DMA,
                pltpu.SemaphoreType.DMA,
            ],
        )(_sc_meta_body)
    return _SC_CACHE["meta"](*args)


def _sc_meta_body(idx0, idx1, idx2, idx3, fdA, fdB, fsA, fsB,
             g0A, g0B, g1A, g1B, g2A, g2B, g3A, g3B,
             ibuf, u0, u1, sem0, sem1):
    c = lax.axis_index("c")
    s = lax.axis_index("s")
    wid = s * NC + c
    jobs = [(idx0, fdA, g0A), (idx0, fdB, g0B),
            (idx1, fdA, g1A), (idx1, fdB, g1B),
            (idx2, fsA, g2A), (idx2, fsB, g2B),
            (idx3, fsA, g3A), (idx3, fsB, g3B)]
    for idx, tbl, out in jobs:
        pltpu.sync_copy(idx.at[pl.ds(wid * MCH, MCH)], ibuf)
        pltpu.async_copy(tbl.at[ibuf.at[0]], u0, sem0)

        def body(jj, carry, tbl=tbl, out=out):
            pltpu.async_copy(tbl.at[ibuf.at[jj * 2 + 1]], u1, sem1)
            pltpu.make_async_copy(tbl.at[ibuf.at[0]], u0, sem0).wait()
            pltpu.sync_copy(u0, out.at[pl.ds((wid * MCH + jj * 2) * CH, CH)])
            nxt = jnp.minimum(jj * 2 + 2, MCH - 1)
            pltpu.async_copy(tbl.at[ibuf.at[nxt]], u0, sem0)
            pltpu.make_async_copy(tbl.at[ibuf.at[0]], u1, sem1).wait()
            pltpu.sync_copy(u1, out.at[pl.ds((wid * MCH + jj * 2 + 1) * CH, CH)])
            return carry

        lax.fori_loop(0, MCH // 2, body, 0)
        pltpu.make_async_copy(tbl.at[ibuf.at[0]], u0, sem0).wait()


_RB = 1000


def _dot(a, b):
    return jnp.dot(a, b, preferred_element_type=F32)


def _full(shape):
    return pl.BlockSpec(shape, lambda i: (0,) * len(shape))


def _rows(shape):
    return pl.BlockSpec(shape, lambda i: (i,) + (0,) * (len(shape) - 1))


def _nrm(deg):
    return lax.rsqrt(jnp.maximum(deg, 1.0))


def _pre_body(fd, fs, wd, ws, bd, bs, g_dd, g_ds, g_sd, g_ss,
              hd_o, hs_o, xdda, xddb, xdsa, xdsb, xsda, xsdb, xssa, xssb):
    hd = _dot(fd[...], wd[...]) + bd[...]
    hs = _dot(fs[...], ws[...]) + bs[...]
    hd_o[...] = hd
    hs_o[...] = hs
    xdd = hd * _nrm(g_dd[...])
    xds = hd * _nrm(g_ds[...])
    xsd = hs * _nrm(g_sd[...])
    xss = hs * _nrm(g_ss[...])
    xdda[...] = xdd[:, :HALF]
    xddb[...] = xdd[:, HALF:]
    xdsa[...] = xds[:, :HALF]
    xdsb[...] = xds[:, HALF:]
    xsda[...] = xsd[:, :HALF]
    xsdb[...] = xsd[:, HALF:]
    xssa[...] = xss[:, :HALF]
    xssb[...] = xss[:, HALF:]


def _tc_pre(fd, fs, wd, ws, bd, bs, g_dd, g_ds, g_sd, g_ss):
    nblk = N // _RB
    outs = ([jax.ShapeDtypeStruct((N, D), F32)] * 2
            + [jax.ShapeDtypeStruct((N, HALF), F32)] * 8)
    return pl.pallas_call(
        _pre_body,
        grid=(nblk,),
        in_specs=[_rows((_RB, D)), _rows((_RB, D)),
                  _full((D, D)), _full((D, D)),
                  _full((1, D)), _full((1, D)),
                  _rows((_RB, 1)), _rows((_RB, 1)),
                  _rows((_RB, 1)), _rows((_RB, 1))],
        out_specs=[_rows((_RB, D))] * 2 + [_rows((_RB, HALF))] * 8,
        out_shape=outs,
    )(fd, fs, wd, ws, bd, bs, g_dd, g_ds, g_sd, g_ss)


def _post_body_next(adda, addb, asda, asdb, adsa, adsb, assa, assb,
                    w0a, w0b, w1a, w1b, w2a, w2b, w3a, w3b,
                    b01, b23, gi_dd, gi_sd, gi_ds, gi_ss, alpha,
                    g_dd, g_ds, g_sd, g_ss,
                    hd_o, hs_o, xdda, xddb, xdsa, xdsb, xsda, xsdb, xssa, xssb):
    a = alpha[0, 0]
    dn = (_dot(adda[...], w0a[...]) + _dot(addb[...], w0b[...])) * _nrm(gi_dd[...])
    dn = dn + (_dot(asda[...], w1a[...]) + _dot(asdb[...], w1b[...])) * _nrm(gi_sd[...])
    dn = dn + b01[...]
    hd = jnp.where(dn > 0, dn, a * dn)
    sn = (_dot(adsa[...], w2a[...]) + _dot(adsb[...], w2b[...])) * _nrm(gi_ds[...])
    sn = sn + (_dot(assa[...], w3a[...]) + _dot(assb[...], w3b[...])) * _nrm(gi_ss[...])
    sn = sn + b23[...]
    hs = jnp.where(sn > 0, sn, a * sn)
    hd_o[...] = hd
    hs_o[...] = hs
    xdd = hd * _nrm(g_dd[...])
    xds = hd * _nrm(g_ds[...])
    xsd = hs * _nrm(g_sd[...])
    xss = hs * _nrm(g_ss[...])
    xdda[...] = xdd[:, :HALF]
    xddb[...] = xdd[:, HALF:]
    xdsa[...] = xds[:, :HALF]
    xdsb[...] = xds[:, HALF:]
    xsda[...] = xsd[:, :HALF]
    xsdb[...] = xsd[:, HALF:]
    xssa[...] = xss[:, :HALF]
    xssb[...] = xss[:, HALF:]


def _post_body_last(adda, addb, asda, asdb, adsa, adsb, assa, assb,
                    w0a, w0b, w1a, w1b, w2a, w2b, w3a, w3b,
                    b01, b23, gi_dd, gi_sd, gi_ds, gi_ss, alpha,
                    hd_o, hs_o):
    a = alpha[0, 0]
    dn = (_dot(adda[...], w0a[...]) + _dot(addb[...], w0b[...])) * _nrm(gi_dd[...])
    dn = dn + (_dot(asda[...], w1a[...]) + _dot(asdb[...], w1b[...])) * _nrm(gi_sd[...])
    dn = dn + b01[...]
    hd_o[...] = jnp.where(dn > 0, dn, a * dn)
    sn = (_dot(adsa[...], w2a[...]) + _dot(adsb[...], w2b[...])) * _nrm(gi_ds[...])
    sn = sn + (_dot(assa[...], w3a[...]) + _dot(assb[...], w3b[...])) * _nrm(gi_ss[...])
    sn = sn + b23[...]
    hs_o[...] = jnp.where(sn > 0, sn, a * sn)


def _tc_post(has_next, aggs, wslices, b01, b23, gins, alpha, gsrcs):
    nblk = N // _RB
    ins = list(aggs) + list(wslices) + [b01, b23] + list(gins) + [alpha]
    in_specs = ([_rows((_RB, HALF))] * 8 + [_full((HALF, D))] * 8
                + [_full((1, D))] * 2 + [_rows((_RB, 1))] * 4
                + [_full((1, 1))])
    if has_next:
        ins += list(gsrcs)
        in_specs += [_rows((_RB, 1))] * 4
        outs = ([jax.ShapeDtypeStruct((N, D), F32)] * 2
                + [jax.ShapeDtypeStruct((N, HALF), F32)] * 8)
        out_specs = [_rows((_RB, D))] * 2 + [_rows((_RB, HALF))] * 8
        body = _post_body_next
    else:
        outs = [jax.ShapeDtypeStruct((N, D), F32)] * 2
        out_specs = [_rows((_RB, D))] * 2
        body = _post_body_last
    return pl.pallas_call(
        body, grid=(nblk,), in_specs=in_specs, out_specs=out_specs,
        out_shape=outs,
    )(*ins)


def _res_body(hd0, hd1, hd2, hs0, hs1, hs2,
              wd0, wd1, wd2, ws0, ws1, ws2, bd, bs,
              fda, fdb, fsa, fsb):
    fd = (_dot(hd0[...], wd0[...]) + _dot(hd1[...], wd1[...])
          + _dot(hd2[...], wd2[...]) + bd[...])
    fs = (_dot(hs0[...], ws0[...]) + _dot(hs1[...], ws1[...])
          + _dot(hs2[...], ws2[...]) + bs[...])
    fda[...] = fd[:, :HALF]
    fdb[...] = fd[:, HALF:]
    fsa[...] = fs[:, :HALF]
    fsb[...] = fs[:, HALF:]


def _tc_res(hds, hss, wds, wss, bd, bs):
    nblk = N // _RB
    return pl.pallas_call(
        _res_body,
        grid=(nblk,),
        in_specs=[_rows((_RB, D))] * 6 + [_full((D, D))] * 6 + [_full((1, D))] * 2,
        out_specs=[_rows((_RB, HALF))] * 4,
        out_shape=[jax.ShapeDtypeStruct((N, HALF), F32)] * 4,
    )(*hds, *hss, *wds, *wss, bd, bs)


_MB = 512


def _m1_body(g0a, g0b, g1a, g1b, g2a, g2b, g3a, g3b,
             wdd_aa, wdd_ab, wdd_ba, wdd_bb,
             wdr_aa, wdr_ab, wdr_ba, wdr_bb,
             whd_a, whd_b, whs_a, whs_b,
             wm1_a, wm1_b, bm1, wm2, wins, wmlp_a, wmlp_b,
             sc_o, im_o, pr_o):
    bf = jnp.bfloat16
    f0a, f0b = g0a[...], g0b[...]
    f1a, f1b = g1a[...], g1b[...]
    f2a, f2b = g2a[...], g2b[...]
    f3a, f3b = g3a[...], g3b[...]
    p = ((f0a + f1a) * 0.5).astype(bf)
    q = ((f0b + f1b) * 0.5).astype(bf)
    dis_a = ((_dot(p, wdd_aa[...].astype(bf)) + _dot(q, wdd_ba[...].astype(bf))
              + f2a) * 0.5 + f3a) * 0.5
    dis_b = ((_dot(p, wdd_ab[...].astype(bf)) + _dot(q, wdd_bb[...].astype(bf))
              + f2b) * 0.5 + f3b) * 0.5
    p2 = ((f3a + f2a) * 0.5).astype(bf)
    q2 = ((f3b + f2b) * 0.5).astype(bf)
    drug_a = ((_dot(p2, wdr_aa[...].astype(bf)) + _dot(q2, wdr_ba[...].astype(bf))
               + f1a) * 0.5 + f0a) * 0.5
    drug_b = ((_dot(p2, wdr_ab[...].astype(bf)) + _dot(q2, wdr_bb[...].astype(bf))
               + f1b) * 0.5 + f0b) * 0.5
    di = (_dot(drug_a.astype(bf), whd_a[...].astype(bf))
          + _dot(drug_b.astype(bf), whd_b[...].astype(bf)))
    si = (_dot(dis_a.astype(bf), whs_a[...].astype(bf))
          + _dot(dis_b.astype(bf), whs_b[...].astype(bf)))
    t1 = jnp.tanh(_dot(di, wm1_a[...]) + _dot(si, wm1_b[...]) + bm1[...])
    sc_o[...] = _dot(t1, wm2[...])
    im_o[...] = _dot(di, wmlp_a[...]) + _dot(si, wmlp_b[...])
    pr_o[...] = jnp.sum(_dot(di, wins[...]) * si, axis=-1, keepdims=True)


def _tc_m1(gs, wq, wrq, whd, whs, wm1, bm1, wm2, wins, wmlp):
    nblk = (B * BAG) // _MB
    ins = list(gs) + list(wq) + list(wrq) + list(whd) + list(whs) + list(wm1) \
        + [bm1, wm2, wins] + list(wmlp)
    in_specs = ([_rows((_MB, HALF))] * 8
                + [_full((HALF, HALF))] * 8
                + [_full((HALF, HALF))] * 4
                + [_full((HALF, D))] * 2
                + [_full((1, D)), _full((D, 1)), _full((HALF, HALF))]
                + [_full((HALF, 1))] * 2)
    return pl.pallas_call(
        _m1_body, grid=(nblk,), in_specs=in_specs,
        out_specs=[_rows((_MB, 1))] * 3,
        out_shape=[jax.ShapeDtypeStruct((B * BAG, 1), F32)] * 3,
    )(*ins)


def _m2_body(sc_ref, im_ref, pr_ref, out_ref):
    scr = sc_ref[...]
    m = jnp.max(scr, axis=-1, keepdims=True)
    e = jnp.exp(scr - m)
    attn = e / jnp.sum(e, axis=-1, keepdims=True)
    mlp = jnp.sum(attn * im_ref[...], axis=-1, keepdims=True)
    ap = attn * pr_ref[...]
    iota = lax.broadcasted_iota(jnp.int32, ap.shape, 1)
    acc = jnp.zeros((ap.shape[0], 1), F32)
    cur = ap
    for _ in range(KTOP):
        mk = jnp.max(cur, axis=-1, keepdims=True)
        acc = acc + mk
        pos = jnp.min(jnp.where(cur == mk, iota, BAG), axis=-1, keepdims=True)
        cur = jnp.where(iota == pos, -3e38, cur)
    out_ref[...] = (mlp + acc * (1.0 / KTOP)) * 0.5


def _tc_m2(scores, imlp, pred):
    return pl.pallas_call(
        _m2_body, grid=(1,),
        in_specs=[_full((B, BAG))] * 3,
        out_specs=_full((B, 1)),
        out_shape=jax.ShapeDtypeStruct((B, 1), F32),
    )(scores, imlp, pred)


def _split_cols(w):
    return w[:HALF, :], w[HALF:, :]


def kernel(feat_drug, feat_disease, edge_index_drug_drug,
           edge_index_drug_disease, edge_index_disease_drug,
           edge_index_disease_disease, mp_ins,
           W_lin_drug, b_lin_drug, W_lin_dis, b_lin_dis, Wg, bg, a_prelu,
           W_res_drug, b_res_drug, W_res_dis, b_res_dis,
           W_dd, W_dr, W_half_drug, W_half_dis,
           W_mil1, b_mil1, W_mil2, W_ins, W_mlp):
    npad = EPAD - E
    pad_lo = (jnp.arange(npad, dtype=jnp.int32) % 16)
    pad_hi = pad_lo + N

    def pad_edges(ei):
        src = jnp.concatenate([ei[0], pad_lo]).reshape(NCH, CH)
        dst = jnp.concatenate([ei[1], pad_hi]).reshape(NCH, CH)
        srcd = jnp.concatenate([ei[0], pad_hi]).reshape(NCH, CH)
        dstd = jnp.concatenate([ei[1], pad_hi]).reshape(NCH, CH)
        return src, dst, srcd, dstd

    dd = pad_edges(edge_index_drug_drug)
    ds = pad_edges(edge_index_drug_disease)
    sd = pad_edges(edge_index_disease_drug)
    ss = pad_edges(edge_index_disease_disease)

    ones128 = jnp.ones((CH,), F32)
    zer640 = jnp.zeros((STRIP,), F32)
    zer_blk = jnp.zeros((STRIP, HALF), F32)

    degs = _sc_degrees(ones128, zer640,
                       dd[2], dd[3], ds[2], ds[3], sd[2], sd[3], ss[2], ss[3])
    (g_dd_s, g_dd_d, g_ds_s, g_ds_d,
     g_sd_s, g_sd_d, g_ss_s, g_ss_d) = [g[:N].reshape(N, 1) for g in degs]

    bd = b_lin_drug.reshape(1, D)
    bs = b_lin_dis.reshape(1, D)
    pre = _tc_pre(feat_drug, feat_disease, W_lin_drug, W_lin_dis, bd, bs,
                  g_dd_s, g_ds_s, g_sd_s, g_ss_s)
    hd0, hs0 = pre[0], pre[1]
    tables = pre[2:]

    hs_list = [hd0]
    hss_list = [hs0]
    hd_cur, hs_cur = hd0, hs0
    gins = (g_dd_d, g_sd_d, g_ds_d, g_ss_d)
    gsrcs = (g_dd_s, g_ds_s, g_sd_s, g_ss_s)
    for layer in range(2):
        xdda, xddb, xdsa, xdsb, xsda, xsdb, xssa, xssb = tables
        aggs8 = _sc_agg(zer_blk,
                        dd[0], dd[1], sd[0], sd[1], ds[0], ds[1], ss[0], ss[1],
                        xdda, xddb, xsda, xsdb, xdsa, xdsb, xssa, xssb)
        agg = [a[:N, :] for a in aggs8]
        wsl = []
        for r in range(4):
            wa, wb = _split_cols(Wg[layer, r])
            wsl += [wa, wb]
        b01 = (bg[layer, 0] + bg[layer, 1]).reshape(1, D)
        b23 = (bg[layer, 2] + bg[layer, 3]).reshape(1, D)
        alpha = a_prelu[layer].reshape(1, 1)
        has_next = layer + 1 < 2
        post = _tc_post(has_next, agg, wsl, b01, b23, gins, alpha, gsrcs)
        hd_cur, hs_cur = post[0], post[1]
        hs_list.append(hd_cur)
        hss_list.append(hs_cur)
        if has_next:
            tables = post[2:]

    wd_sl = [W_res_drug[i * D:(i + 1) * D, :] for i in range(3)]
    ws_sl = [W_res_dis[i * D:(i + 1) * D, :] for i in range(3)]
    fda, fdb, fsa, fsb = _tc_res(hs_list, hss_list, wd_sl, ws_sl,
                                 b_res_drug.reshape(1, D),
                                 b_res_dis.reshape(1, D))

    idx = [mp_ins[:, :, j].reshape((B * BAG) // CH, CH) for j in range(4)]
    gs = _sc_meta(idx[0], idx[1], idx[2], idx[3], fda, fdb, fsa, fsb)

    wq = [W_dd[:HALF, :HALF], W_dd[:HALF, HALF:],
          W_dd[HALF:, :HALF], W_dd[HALF:, HALF:]]
    wrq = [W_dr[:HALF, :HALF], W_dr[:HALF, HALF:],
           W_dr[HALF:, :HALF], W_dr[HALF:, HALF:]]
    whd = _split_cols(W_half_drug)
    whs = _split_cols(W_half_dis)
    wm1 = _split_cols(W_mil1)
    wmlp = _split_cols(W_mlp)
    scores, imlp, pred = _tc_m1(gs, wq, wrq, whd, whs, wm1,
                                b_mil1.reshape(1, D), W_mil2, W_ins, wmlp)
    return _tc_m2(scores.reshape(B, BAG), imlp.reshape(B, BAG),
                  pred.reshape(B, BAG))
